# baseline (speedup 1.0000x reference)
import functools

import jax
import jax.numpy as jnp
from jax.experimental import pallas as pl
from jax.experimental.pallas import tpu as pltpu


def _width_interp_matrix(out_size, in_size):
    if in_size == 1:
        return jnp.ones((1, out_size), dtype=jnp.float32)
    scale = (in_size - 1) / (out_size - 1)
    src = jnp.arange(out_size, dtype=jnp.float32) * scale
    i0 = jnp.clip(jnp.floor(src).astype(jnp.int32), 0, in_size - 1)
    i1 = jnp.clip(i0 + 1, 0, in_size - 1)
    w1 = src - i0.astype(jnp.float32)
    w0 = 1.0 - w1
    oh0 = jax.nn.one_hot(i0, in_size, dtype=jnp.float32)
    oh1 = jax.nn.one_hot(i1, in_size, dtype=jnp.float32)
    return (w0[:, None] * oh0 + w1[:, None] * oh1).T


def _copy_pair(scratch_ref, o_ref, sems, slot, blk, tb):
    half = tb // 2
    c0 = pltpu.make_async_copy(
        scratch_ref.at[slot, pl.ds(0, half)],
        o_ref.at[pl.ds(blk * tb, half)],
        sems.at[slot, 0])
    c1 = pltpu.make_async_copy(
        scratch_ref.at[slot, pl.ds(half, half)],
        o_ref.at[pl.ds(blk * tb + half, half)],
        sems.at[slot, 1])
    return c0, c1


def _upsample_kernel(h, nbuf, wxt_ref, x_ref, o_ref, scratch_ref, sems):
    i = pl.program_id(0)
    g = pl.num_programs(0)
    x = x_ref[...]
    tb, _, w = x.shape
    wout = wxt_ref.shape[1]
    s = 2 * h - 1
    slot = jax.lax.rem(i, nbuf)

    @pl.when(i >= nbuf)
    def _():
        c0, c1 = _copy_pair(scratch_ref, o_ref, sems, slot, i - nbuf, tb)
        c0.wait()
        c1.wait()

    t = jnp.dot(x.reshape(tb * h, w).astype(jnp.bfloat16), wxt_ref[...],
                preferred_element_type=jnp.float32).reshape(tb, h, wout)

    td = pltpu.roll(t, 1, 1)
    tu = pltpu.roll(t, h - 1, 1)

    k = jax.lax.broadcasted_iota(jnp.int32, (tb, h, wout), 1).astype(
        jnp.float32)
    inv_s = jnp.float32(1.0 / s)
    a = k * inv_s
    even = t + a * (td - t)
    d = (jnp.float32(h - 1) - k) * inv_s
    odd = t + d * (tu - t)

    buf = scratch_ref.at[slot]
    buf[:, pl.ds(0, h, 2), :] = even.astype(o_ref.dtype)
    buf[:, pl.ds(1, h, 2), :] = odd.astype(o_ref.dtype)

    c0, c1 = _copy_pair(scratch_ref, o_ref, sems, slot, i, tb)
    c0.start()
    c1.start()

    @pl.when(i == g - 1)
    def _():
        for sl in range(nbuf):
            blk = i - jax.lax.rem(jnp.int32(i) - sl, nbuf)
            d0, d1 = _copy_pair(scratch_ref, o_ref, sems, sl, blk, tb)
            d0.wait()
            d1.wait()


@jax.jit
def _up2x(x):
    N, C, H, W = x.shape
    Hout, Wout = 2 * H, 2 * W
    B = N * C
    out_dtype = x.dtype
    in_b = x.dtype.itemsize
    nbuf = 2

    tb = 256
    while B % tb != 0 or B // tb < 2 * nbuf:
        tb //= 2
    g = B // tb

    wxt = _width_interp_matrix(Wout, W).astype(jnp.bfloat16)
    x3 = x.reshape(B, H, W)
    cost = pl.CostEstimate(
        flops=2 * B * H * W * Wout + 8 * B * H * Wout,
        transcendentals=0,
        bytes_accessed=int(B * H * W * in_b * 5),
    )
    out = pl.pallas_call(
        functools.partial(_upsample_kernel, H, nbuf),
        out_shape=jax.ShapeDtypeStruct((B, Hout, Wout), out_dtype),
        grid=(g,),
        in_specs=[
            pl.BlockSpec((W, Wout), lambda i: (0, 0)),
            pl.BlockSpec((tb, H, W), lambda i: (i, 0, 0)),
        ],
        out_specs=pl.BlockSpec(memory_space=pltpu.MemorySpace.HBM),
        scratch_shapes=[
            pltpu.VMEM((nbuf, tb, Hout, Wout), out_dtype),
            pltpu.SemaphoreType.DMA((nbuf, 2)),
        ],
        compiler_params=pltpu.CompilerParams(
            dimension_semantics=("arbitrary",),
            vmem_limit_bytes=56 * 1024 * 1024),
        cost_estimate=cost,
    )(wxt, x3)
    return out.reshape(N, C, Hout, Wout)


def kernel(x):
    return _up2x(x)


# speedup vs baseline: 1.0118x; 1.0118x over previous
import functools

import jax
import jax.numpy as jnp
from jax.experimental import pallas as pl
from jax.experimental.pallas import tpu as pltpu


def _width_interp_matrix(out_size, in_size):
    if in_size == 1:
        return jnp.ones((1, out_size), dtype=jnp.float32)
    scale = (in_size - 1) / (out_size - 1)
    src = jnp.arange(out_size, dtype=jnp.float32) * scale
    i0 = jnp.clip(jnp.floor(src).astype(jnp.int32), 0, in_size - 1)
    i1 = jnp.clip(i0 + 1, 0, in_size - 1)
    w1 = src - i0.astype(jnp.float32)
    w0 = 1.0 - w1
    oh0 = jax.nn.one_hot(i0, in_size, dtype=jnp.float32)
    oh1 = jax.nn.one_hot(i1, in_size, dtype=jnp.float32)
    return (w0[:, None] * oh0 + w1[:, None] * oh1).T


def _upsample_kernel(h, wxt_ref, x_ref, o_ref):
    x = x_ref[...]
    tb, _, w = x.shape
    wout = wxt_ref.shape[1]
    s = 2 * h - 1

    t = jnp.dot(x.reshape(tb * h, w).astype(jnp.bfloat16), wxt_ref[...],
                preferred_element_type=jnp.float32).reshape(tb, h, wout)

    td = pltpu.roll(t, 1, 1)
    tu = pltpu.roll(t, h - 1, 1)

    k = jax.lax.broadcasted_iota(jnp.int32, (tb, h, wout), 1).astype(
        jnp.float32)
    inv_s = jnp.float32(1.0 / s)
    a = k * inv_s
    even = t + a * (td - t)
    d = (jnp.float32(h - 1) - k) * inv_s
    odd = t + d * (tu - t)

    o_ref[:, pl.ds(0, h, 2), :] = even.astype(o_ref.dtype)
    o_ref[:, pl.ds(1, h, 2), :] = odd.astype(o_ref.dtype)


@jax.jit
def _up2x(x):
    N, C, H, W = x.shape
    Hout, Wout = 2 * H, 2 * W
    B = N * C
    out_dtype = x.dtype
    in_b = x.dtype.itemsize

    tb = 256
    while B % tb != 0 or B // tb < 2:
        tb //= 2
    g = B // tb

    wxt = _width_interp_matrix(Wout, W).astype(jnp.bfloat16)
    x3 = x.reshape(B, H, W)
    cost = pl.CostEstimate(
        flops=2 * B * 2 * H * W * Wout + 8 * B * H * W,
        transcendentals=0,
        bytes_accessed=int(B * H * W * in_b * 5),
    )
    out = pl.pallas_call(
        functools.partial(_upsample_kernel, H),
        out_shape=jax.ShapeDtypeStruct((B, Hout, Wout), out_dtype),
        grid=(g,),
        in_specs=[
            pl.BlockSpec((W, Wout), lambda i: (0, 0)),
            pl.BlockSpec((tb, H, W), lambda i: (i, 0, 0)),
        ],
        out_specs=pl.BlockSpec((tb, Hout, Wout), lambda i: (i, 0, 0)),
        compiler_params=pltpu.CompilerParams(
            dimension_semantics=("parallel",),
            vmem_limit_bytes=56 * 1024 * 1024),
        cost_estimate=cost,
    )(wxt, x3)
    return out.reshape(N, C, Hout, Wout)


def kernel(x):
    return _up2x(x)


# speedup vs baseline: 1.0129x; 1.0011x over previous
import functools

import jax
import jax.numpy as jnp
from jax.experimental import pallas as pl
from jax.experimental.pallas import tpu as pltpu


def _width_interp_matrix(out_size, in_size):
    if in_size == 1:
        return jnp.ones((1, out_size), dtype=jnp.float32)
    scale = (in_size - 1) / (out_size - 1)
    src = jnp.arange(out_size, dtype=jnp.float32) * scale
    i0 = jnp.clip(jnp.floor(src).astype(jnp.int32), 0, in_size - 1)
    i1 = jnp.clip(i0 + 1, 0, in_size - 1)
    w1 = src - i0.astype(jnp.float32)
    w0 = 1.0 - w1
    oh0 = jax.nn.one_hot(i0, in_size, dtype=jnp.float32)
    oh1 = jax.nn.one_hot(i1, in_size, dtype=jnp.float32)
    return (w0[:, None] * oh0 + w1[:, None] * oh1).T


def _upsample_kernel(h, wxt_ref, x_ref, o_ref):
    x = x_ref[...]
    tb, _, w = x.shape
    wout = wxt_ref.shape[1]
    s = 2 * h - 1

    t = jnp.dot(x.reshape(tb * h, w).astype(jnp.bfloat16), wxt_ref[...],
                preferred_element_type=jnp.float32).reshape(tb, h, wout)

    td = pltpu.roll(t, 1, 1)
    tu = pltpu.roll(t, h - 1, 1)

    k = jax.lax.broadcasted_iota(jnp.int32, (tb, h, wout), 1).astype(
        jnp.float32)
    inv_s = jnp.float32(1.0 / s)
    a = k * inv_s
    even = t + a * (td - t)
    d = (jnp.float32(h - 1) - k) * inv_s
    odd = t + d * (tu - t)

    o_ref[:, pl.ds(0, h, 2), :] = even.astype(o_ref.dtype)
    o_ref[:, pl.ds(1, h, 2), :] = odd.astype(o_ref.dtype)


@jax.jit
def _up2x(x):
    N, C, H, W = x.shape
    Hout, Wout = 2 * H, 2 * W
    B = N * C
    out_dtype = x.dtype
    in_b = x.dtype.itemsize

    tb = 256
    while B % tb != 0 or B // tb < 2:
        tb //= 2
    g = B // tb

    wxt = _width_interp_matrix(Wout, W).astype(jnp.bfloat16)
    x3 = x.reshape(B, H, W)
    cost = pl.CostEstimate(
        flops=2 * B * H * W * Wout + 8 * B * H * Wout,
        transcendentals=0,
        bytes_accessed=int(B * H * W * in_b * 5),
    )
    out = pl.pallas_call(
        functools.partial(_upsample_kernel, H),
        out_shape=jax.ShapeDtypeStruct((B, Hout, Wout), out_dtype),
        grid=(g,),
        in_specs=[
            pl.BlockSpec((W, Wout), lambda i: (0, 0)),
            pl.BlockSpec((tb, H, W), lambda i: (i, 0, 0)),
        ],
        out_specs=pl.BlockSpec((tb, Hout, Wout), lambda i: (i, 0, 0)),
        compiler_params=pltpu.CompilerParams(
            dimension_semantics=("parallel",),
            vmem_limit_bytes=56 * 1024 * 1024),
        cost_estimate=cost,
    )(wxt, x3)
    return out.reshape(N, C, Hout, Wout)


def kernel(x):
    return _up2x(x)
